# Initial kernel scaffold; baseline (speedup 1.0000x reference)
import jax
import jax.numpy as jnp
from jax.experimental import pallas as pl


def kernel(rew_matrix, state):
    raise NotImplementedError("write your pallas kernel here")


# speedup vs baseline: 135.5156x; 135.5156x over previous
import functools

import jax
import jax.numpy as jnp
from jax import lax
from jax.experimental import pallas as pl
from jax.experimental.pallas import tpu as pltpu
from jax.experimental.pallas import tpu_sc as plsc

_N_WORKERS = 32
_CHUNK = 12800


@functools.partial(jax.jit, static_argnames=("total",))
def _sc_gather(table, flat_idx, total):
    per_worker = total // _N_WORKERS
    n_chunks = per_worker // _CHUNK
    mesh = plsc.VectorSubcoreMesh(core_axis_name="c", subcore_axis_name="s")

    @functools.partial(
        pl.kernel,
        mesh=mesh,
        out_type=jax.ShapeDtypeStruct((total,), jnp.float32),
        scratch_types=[
            pltpu.VMEM((_CHUNK,), jnp.int32),
            pltpu.VMEM((_CHUNK,), jnp.float32),
            pltpu.SemaphoreType.DMA,
        ],
    )
    def k(table_hbm, idx_hbm, out_hbm, idx_v, rows_v, sem):
        wid = lax.axis_index("s") * 2 + lax.axis_index("c")
        base = wid * per_worker

        def body(i, _):
            off = base + i * _CHUNK
            pltpu.sync_copy(idx_hbm.at[pl.ds(off, _CHUNK)], idx_v)
            pltpu.async_copy(table_hbm.at[idx_v], rows_v, sem).wait()
            pltpu.sync_copy(rows_v, out_hbm.at[pl.ds(off, _CHUNK)])
            return 0

        lax.fori_loop(0, n_chunks, body, 0)

    return k(table, flat_idx)


def kernel(rew_matrix, state):
    flat = state.reshape(-1)
    out = _sc_gather(rew_matrix, flat, flat.shape[0])
    return out.reshape(state.shape)


# speedup vs baseline: 141.1377x; 1.0415x over previous
import functools

import jax
import jax.numpy as jnp
from jax import lax
from jax.experimental import pallas as pl
from jax.experimental.pallas import tpu as pltpu
from jax.experimental.pallas import tpu_sc as plsc

_N_WORKERS = 32
_CHUNK = 51200


@functools.partial(jax.jit, static_argnames=("total",))
def _sc_gather(table, flat_idx, total):
    per_worker = total // _N_WORKERS
    n_chunks = per_worker // _CHUNK
    mesh = plsc.VectorSubcoreMesh(core_axis_name="c", subcore_axis_name="s")

    @functools.partial(
        pl.kernel,
        mesh=mesh,
        out_type=jax.ShapeDtypeStruct((total,), jnp.float32),
        scratch_types=[
            pltpu.VMEM((_CHUNK,), jnp.int32),
            pltpu.VMEM((_CHUNK,), jnp.float32),
            pltpu.SemaphoreType.DMA,
        ],
    )
    def k(table_hbm, idx_hbm, out_hbm, idx_v, rows_v, sem):
        wid = lax.axis_index("s") * 2 + lax.axis_index("c")
        base = wid * per_worker

        def body(i, _):
            off = base + i * _CHUNK
            pltpu.sync_copy(idx_hbm.at[pl.ds(off, _CHUNK)], idx_v)
            pltpu.async_copy(table_hbm.at[idx_v], rows_v, sem).wait()
            pltpu.sync_copy(rows_v, out_hbm.at[pl.ds(off, _CHUNK)])
            return 0

        lax.fori_loop(0, n_chunks, body, 0)

    return k(table, flat_idx)


def kernel(rew_matrix, state):
    flat = state.reshape(-1)
    out = _sc_gather(rew_matrix, flat, flat.shape[0])
    return out.reshape(state.shape)


# speedup vs baseline: 215.5872x; 1.5275x over previous
import functools

import jax
import jax.numpy as jnp
from jax import lax
from jax.experimental import pallas as pl
from jax.experimental.pallas import tpu as pltpu
from jax.experimental.pallas import tpu_sc as plsc

_N_WORKERS = 32
_CHUNK = 25600
_N_STATES = 1000000
_STAGE_SEG = 25000


@functools.partial(jax.jit, static_argnames=("total",))
def _sc_gather(table, flat_idx, total):
    per_worker = total // _N_WORKERS
    n_chunks = per_worker // _CHUNK
    mesh = plsc.VectorSubcoreMesh(core_axis_name="c", subcore_axis_name="s")

    @functools.partial(
        pl.kernel,
        mesh=mesh,
        out_type=jax.ShapeDtypeStruct((total,), jnp.float32),
        scratch_types=[
            pltpu.VMEM((_CHUNK,), jnp.int32),
            pltpu.VMEM((_CHUNK,), jnp.float32),
            pltpu.VMEM_SHARED((_N_STATES,), jnp.float32),
            pltpu.SemaphoreType.DMA,
        ],
    )
    def k(table_hbm, idx_hbm, out_hbm, idx_v, rows_v, table_sp, sem):
        sid = lax.axis_index("s")
        wid = sid * 2 + lax.axis_index("c")
        base = wid * per_worker

        n_pieces = _N_STATES // _STAGE_SEG
        for p in range(3):
            piece = sid + p * 16

            @pl.when(piece < n_pieces)
            def _stage():
                seg = pl.ds(piece * _STAGE_SEG, _STAGE_SEG)
                stage_v = rows_v.at[pl.ds(0, _STAGE_SEG)]
                pltpu.sync_copy(table_hbm.at[seg], stage_v)
                pltpu.sync_copy(stage_v, table_sp.at[seg])

        plsc.subcore_barrier()

        def body(i, _):
            off = base + i * _CHUNK
            pltpu.sync_copy(idx_hbm.at[pl.ds(off, _CHUNK)], idx_v)
            pltpu.async_copy(table_sp.at[idx_v], rows_v, sem).wait()
            pltpu.sync_copy(rows_v, out_hbm.at[pl.ds(off, _CHUNK)])
            return 0

        lax.fori_loop(0, n_chunks, body, 0)

    return k(table, flat_idx)


def kernel(rew_matrix, state):
    flat = state.reshape(-1)
    out = _sc_gather(rew_matrix, flat, flat.shape[0])
    return out.reshape(state.shape)


# speedup vs baseline: 226.5857x; 1.0510x over previous
import functools

import jax
import jax.numpy as jnp
from jax import lax
from jax.experimental import pallas as pl
from jax.experimental.pallas import tpu as pltpu
from jax.experimental.pallas import tpu_sc as plsc

_N_WORKERS = 32
_CHUNK = 12800
_HBM_SPLIT = 0
_N_STATES = 1000000
_STAGE_SEG = 10000


@functools.partial(jax.jit, static_argnames=("total",))
def _sc_gather(table, flat_idx, total):
    per_worker = total // _N_WORKERS
    n_chunks = per_worker // _CHUNK
    mesh = plsc.VectorSubcoreMesh(core_axis_name="c", subcore_axis_name="s")

    @functools.partial(
        pl.kernel,
        mesh=mesh,
        out_type=jax.ShapeDtypeStruct((total,), jnp.float32),
        scratch_types=[
            pltpu.VMEM((_CHUNK,), jnp.int32),
            pltpu.VMEM((_CHUNK,), jnp.int32),
            pltpu.VMEM((_CHUNK,), jnp.float32),
            pltpu.VMEM((_CHUNK,), jnp.float32),
            pltpu.VMEM_SHARED((_N_STATES,), jnp.float32),
            pltpu.SemaphoreType.DMA,
            pltpu.SemaphoreType.DMA,
            pltpu.SemaphoreType.DMA,
            pltpu.SemaphoreType.DMA,
            pltpu.SemaphoreType.DMA,
            pltpu.SemaphoreType.DMA,
        ],
    )
    def k(table_hbm, idx_hbm, out_hbm, idx_v0, idx_v1, rows_v0, rows_v1,
          table_sp, sem_i0, sem_i1, sem_g0, sem_g1, sem_o0, sem_o1):
        idx_v = (idx_v0, idx_v1)
        rows_v = (rows_v0, rows_v1)
        sid = lax.axis_index("s")
        wid = sid * 2 + lax.axis_index("c")
        base = wid * per_worker
        sem_i = (sem_i0, sem_i1)
        sem_o = (sem_o0, sem_o1)

        def idx_load(i):
            off = base + i * _CHUNK
            return pltpu.async_copy(
                idx_hbm.at[pl.ds(off, _CHUNK)], idx_v[i % 2], sem_i[i % 2]
            )

        loads = {0: idx_load(0)}
        if n_chunks > 1:
            loads[1] = idx_load(1)

        n_pieces = _N_STATES // _STAGE_SEG
        n_rounds = -(-n_pieces // 16)
        for p in range(n_rounds):
            piece = p * 16 + sid

            @pl.when(piece < n_pieces)
            def _stage():
                seg = pl.ds(piece * _STAGE_SEG, _STAGE_SEG)
                stage_v = rows_v[0].at[pl.ds(0, _STAGE_SEG)]
                pltpu.sync_copy(table_hbm.at[seg], stage_v)
                pltpu.sync_copy(stage_v, table_sp.at[seg])

        plsc.subcore_barrier()

        stores = {}
        for i in range(n_chunks):
            b = i % 2
            loads[i].wait()
            if i - 2 in stores:
                stores[i - 2].wait()
            if _HBM_SPLIT > 0:
                g_h = pltpu.async_copy(
                    table_hbm.at[idx_v[b].at[pl.ds(0, _HBM_SPLIT)]],
                    rows_v[b].at[pl.ds(0, _HBM_SPLIT)],
                    sem_g0,
                )
                g_s = pltpu.async_copy(
                    table_sp.at[idx_v[b].at[pl.ds(_HBM_SPLIT,
                                                    _CHUNK - _HBM_SPLIT)]],
                    rows_v[b].at[pl.ds(_HBM_SPLIT, _CHUNK - _HBM_SPLIT)],
                    sem_g1,
                )
                g_h.wait()
                g_s.wait()
            else:
                pltpu.async_copy(
                    table_sp.at[idx_v[b]], rows_v[b], sem_g0
                ).wait()
            if i + 2 < n_chunks:
                loads[i + 2] = idx_load(i + 2)
            stores[i] = pltpu.async_copy(
                rows_v[b], out_hbm.at[pl.ds(base + i * _CHUNK, _CHUNK)],
                sem_o[b],
            )
        for i in (n_chunks - 2, n_chunks - 1):
            if i in stores:
                stores[i].wait()

    return k(table, flat_idx)


def kernel(rew_matrix, state):
    flat = state.reshape(-1)
    out = _sc_gather(rew_matrix, flat, flat.shape[0])
    return out.reshape(state.shape)
